# baseline (speedup 1.0000x reference)
import functools

import jax
import jax.numpy as jnp
from jax import lax
from jax.experimental import pallas as pl
from jax.experimental.pallas import tpu as pltpu

B = 16384
N = 50
F = 16
OBS = 48
H = 64
A = 5

BB = 128


def _fused_kernel(obs_ref, node_ref, adj_ref, aid_ref,
                  w1_ref, b1_ref, w2_ref, b2_ref,
                  wm1o_ref, wm1n_ref, bm1_ref, wm2_ref, bm2_ref,
                  wa_ref, ba_ref,
                  act_ref, alp_ref):
    adj = adj_ref[...]
    node = node_ref[...]

    adj_n = adj / (jnp.sum(adj, axis=-1, keepdims=True) + 1e-6)

    h1 = lax.dot_general(node, w1_ref[...], (((2,), (0,)), ((), ())),
                         preferred_element_type=jnp.float32)
    h1 = jnp.maximum(h1 + b1_ref[...], 0.0)
    g1 = lax.dot_general(adj_n, h1, (((2,), (1,)), ((0,), (0,))),
                         preferred_element_type=jnp.float32)

    h2 = lax.dot_general(g1, w2_ref[...], (((2,), (0,)), ((), ())),
                         preferred_element_type=jnp.float32)
    h2 = jnp.maximum(h2 + b2_ref[...], 0.0)

    aid = aid_ref[...]
    n_iota = lax.broadcasted_iota(jnp.int32, (BB, N), 1)
    onehot = (n_iota == aid).astype(jnp.float32)
    egon = jnp.sum(adj_n * onehot[:, :, None], axis=1)
    nbd = lax.dot_general(egon, h2, (((1,), (1,)), ((0,), (0,))),
                          preferred_element_type=jnp.float32)

    x = jnp.dot(obs_ref[...], wm1o_ref[...],
                preferred_element_type=jnp.float32)
    x += jnp.dot(nbd, wm1n_ref[...], preferred_element_type=jnp.float32)
    x = jnp.maximum(x + bm1_ref[...], 0.0)
    x = jnp.maximum(jnp.dot(x, wm2_ref[...],
                            preferred_element_type=jnp.float32)
                    + bm2_ref[...], 0.0)

    logits = jnp.dot(x, wa_ref[...],
                     preferred_element_type=jnp.float32) + ba_ref[...]
    m = jnp.max(logits, axis=-1, keepdims=True)
    lse0 = jnp.log(jnp.sum(jnp.exp(logits - m), axis=-1, keepdims=True))
    a_iota = lax.broadcasted_iota(jnp.int32, (BB, A), 1)
    cand = jnp.where(logits == m, a_iota, A)
    act_ref[...] = jnp.min(cand, axis=-1, keepdims=True)
    alp_ref[...] = 0.0 - lse0


@jax.jit
def kernel(obs, node_obs, adj, agent_id,
           W_gnn1, b_gnn1, W_gnn2, b_gnn2,
           W_mlp1, b_mlp1, W_mlp2, b_mlp2,
           W_act, b_act):
    w_mlp1_obs = W_mlp1[:OBS]
    w_mlp1_nbd = W_mlp1[OBS:]
    b1 = b_gnn1.reshape(1, 1, H)
    b2 = b_gnn2.reshape(1, 1, H)
    bm1 = b_mlp1.reshape(1, H)
    bm2 = b_mlp2.reshape(1, H)
    ba = b_act.reshape(1, A)

    grid = (B // BB,)
    actions, alp = pl.pallas_call(
        _fused_kernel,
        grid=grid,
        in_specs=[
            pl.BlockSpec((BB, OBS), lambda i: (i, 0)),
            pl.BlockSpec((BB, N, F), lambda i: (i, 0, 0)),
            pl.BlockSpec((BB, N, N), lambda i: (i, 0, 0)),
            pl.BlockSpec((BB, 1), lambda i: (i, 0)),
            pl.BlockSpec((F, H), lambda i: (0, 0)),
            pl.BlockSpec((1, 1, H), lambda i: (0, 0, 0)),
            pl.BlockSpec((H, H), lambda i: (0, 0)),
            pl.BlockSpec((1, 1, H), lambda i: (0, 0, 0)),
            pl.BlockSpec((OBS, H), lambda i: (0, 0)),
            pl.BlockSpec((H, H), lambda i: (0, 0)),
            pl.BlockSpec((1, H), lambda i: (0, 0)),
            pl.BlockSpec((H, H), lambda i: (0, 0)),
            pl.BlockSpec((1, H), lambda i: (0, 0)),
            pl.BlockSpec((H, A), lambda i: (0, 0)),
            pl.BlockSpec((1, A), lambda i: (0, 0)),
        ],
        out_specs=[
            pl.BlockSpec((BB, 1), lambda i: (i, 0)),
            pl.BlockSpec((BB, 1), lambda i: (i, 0)),
        ],
        out_shape=[
            jax.ShapeDtypeStruct((B, 1), jnp.int32),
            jax.ShapeDtypeStruct((B, 1), jnp.float32),
        ],
        compiler_params=pltpu.CompilerParams(
            dimension_semantics=("arbitrary",),
        ),
    )(obs, node_obs, adj, agent_id,
      W_gnn1, b1, W_gnn2, b2,
      w_mlp1_obs, w_mlp1_nbd, bm1, W_mlp2, bm2,
      W_act, ba)
    return actions, alp


# speedup vs baseline: 1.2112x; 1.2112x over previous
import functools

import jax
import jax.numpy as jnp
from jax import lax
from jax.experimental import pallas as pl
from jax.experimental.pallas import tpu as pltpu

B = 16384
N = 50
F = 16
OBS = 48
H = 64
A = 5

BB = 128
NP = 56


def _fused_kernel(obs_ref, node_ref, adj_ref, aid_ref,
                  w1_ref, b1_ref, w2_ref, b2_ref,
                  wm1o_ref, wm1n_ref, bm1_ref, wm2_ref, bm2_ref,
                  wa_ref, ba_ref,
                  act_ref, alp_ref):
    adj = adj_ref[...]
    node = node_ref[...]

    adj_n = adj / (jnp.sum(adj, axis=-1, keepdims=True) + 1e-6)

    h1 = lax.dot_general(node, w1_ref[...], (((2,), (0,)), ((), ())),
                         preferred_element_type=jnp.float32)
    h1 = jnp.maximum(h1 + b1_ref[...], 0.0)
    g1 = lax.dot_general(adj_n, h1, (((2,), (1,)), ((0,), (0,))),
                         preferred_element_type=jnp.float32)

    h2 = lax.dot_general(g1, w2_ref[...], (((2,), (0,)), ((), ())),
                         preferred_element_type=jnp.float32)
    h2 = jnp.maximum(h2 + b2_ref[...], 0.0)

    aid = aid_ref[...]
    n_iota = lax.broadcasted_iota(jnp.int32, (BB, NP), 1)
    onehot = (n_iota == aid).astype(jnp.float32)
    egon = jnp.sum(adj_n * onehot[:, :, None], axis=1)
    nbd = lax.dot_general(egon, h2, (((1,), (1,)), ((0,), (0,))),
                          preferred_element_type=jnp.float32)

    x = jnp.dot(obs_ref[...], wm1o_ref[...],
                preferred_element_type=jnp.float32)
    x += jnp.dot(nbd, wm1n_ref[...], preferred_element_type=jnp.float32)
    x = jnp.maximum(x + bm1_ref[...], 0.0)
    x = jnp.maximum(jnp.dot(x, wm2_ref[...],
                            preferred_element_type=jnp.float32)
                    + bm2_ref[...], 0.0)

    logits = jnp.dot(x, wa_ref[...],
                     preferred_element_type=jnp.float32) + ba_ref[...]
    m = jnp.max(logits, axis=-1, keepdims=True)
    lse0 = jnp.log(jnp.sum(jnp.exp(logits - m), axis=-1, keepdims=True))
    a_iota = lax.broadcasted_iota(jnp.int32, (BB, A), 1)
    cand = jnp.where(logits == m, a_iota, A)
    act_ref[...] = jnp.min(cand, axis=-1, keepdims=True)
    alp_ref[...] = 0.0 - lse0


@jax.jit
def kernel(obs, node_obs, adj, agent_id,
           W_gnn1, b_gnn1, W_gnn2, b_gnn2,
           W_mlp1, b_mlp1, W_mlp2, b_mlp2,
           W_act, b_act):
    adj_p = jnp.pad(adj, ((0, 0), (0, NP - N), (0, NP - N)))
    node_p = jnp.pad(node_obs, ((0, 0), (0, NP - N), (0, 0)))
    w_mlp1_obs = W_mlp1[:OBS]
    w_mlp1_nbd = W_mlp1[OBS:]
    b1 = b_gnn1.reshape(1, 1, H)
    b2 = b_gnn2.reshape(1, 1, H)
    bm1 = b_mlp1.reshape(1, H)
    bm2 = b_mlp2.reshape(1, H)
    ba = b_act.reshape(1, A)

    grid = (B // BB,)
    actions, alp = pl.pallas_call(
        _fused_kernel,
        grid=grid,
        in_specs=[
            pl.BlockSpec((BB, OBS), lambda i: (i, 0)),
            pl.BlockSpec((BB, NP, F), lambda i: (i, 0, 0)),
            pl.BlockSpec((BB, NP, NP), lambda i: (i, 0, 0)),
            pl.BlockSpec((BB, 1), lambda i: (i, 0)),
            pl.BlockSpec((F, H), lambda i: (0, 0)),
            pl.BlockSpec((1, 1, H), lambda i: (0, 0, 0)),
            pl.BlockSpec((H, H), lambda i: (0, 0)),
            pl.BlockSpec((1, 1, H), lambda i: (0, 0, 0)),
            pl.BlockSpec((OBS, H), lambda i: (0, 0)),
            pl.BlockSpec((H, H), lambda i: (0, 0)),
            pl.BlockSpec((1, H), lambda i: (0, 0)),
            pl.BlockSpec((H, H), lambda i: (0, 0)),
            pl.BlockSpec((1, H), lambda i: (0, 0)),
            pl.BlockSpec((H, A), lambda i: (0, 0)),
            pl.BlockSpec((1, A), lambda i: (0, 0)),
        ],
        out_specs=[
            pl.BlockSpec((BB, 1), lambda i: (i, 0)),
            pl.BlockSpec((BB, 1), lambda i: (i, 0)),
        ],
        out_shape=[
            jax.ShapeDtypeStruct((B, 1), jnp.int32),
            jax.ShapeDtypeStruct((B, 1), jnp.float32),
        ],
        compiler_params=pltpu.CompilerParams(
            dimension_semantics=("arbitrary",),
        ),
    )(obs, node_p, adj_p, agent_id,
      W_gnn1, b1, W_gnn2, b2,
      w_mlp1_obs, w_mlp1_nbd, bm1, W_mlp2, bm2,
      W_act, ba)
    return actions, alp


# speedup vs baseline: 1.4480x; 1.1955x over previous
import functools

import jax
import jax.numpy as jnp
from jax import lax
from jax.experimental import pallas as pl
from jax.experimental.pallas import tpu as pltpu

B = 16384
N = 50
F = 16
OBS = 48
H = 64
A = 5

BB = 256
NP = 56


def _fused_kernel(obs_ref, node_ref, adj_ref, aid_ref,
                  w1x_ref, b1x_ref, w2x_ref, b2_ref,
                  wm1o_ref, wm1n_ref, bm1_ref, wm2_ref, bm2_ref,
                  wa_ref, ba_ref,
                  act_ref, alp_ref):
    adj = jnp.pad(adj_ref[...], ((0, 0), (0, NP - N), (0, NP - N)))
    node = jnp.pad(node_ref[...], ((0, 0), (0, NP - N), (0, 0)))

    h1x = lax.dot_general(node, w1x_ref[...], (((2,), (0,)), ((), ())),
                          preferred_element_type=jnp.float32)
    h1x = jnp.maximum(h1x + b1x_ref[...], 0.0)
    g1x = lax.dot_general(adj, h1x, (((2,), (1,)), ((0,), (0,))),
                          preferred_element_type=jnp.float32)

    t = lax.dot_general(g1x, w2x_ref[...], (((2,), (0,)), ((), ())),
                        preferred_element_type=jnp.float32)
    inv = 1.0 / (t[:, :, H:] + 1e-6)
    h2 = jnp.maximum(t[:, :, :H] * inv + b2_ref[...], 0.0)

    aid = aid_ref[...]
    n_iota3 = lax.broadcasted_iota(jnp.int32, (BB, NP, NP), 1)
    mask3 = n_iota3 == aid[:, :, None]
    ego = jnp.sum(jnp.where(mask3, adj, 0.0), axis=1)
    inv_e = 1.0 / (jnp.sum(ego, axis=-1, keepdims=True) + 1e-6)
    q = lax.dot_general(ego, h2, (((1,), (1,)), ((0,), (0,))),
                        preferred_element_type=jnp.float32)
    nbd = q * inv_e

    x = jnp.dot(obs_ref[...], wm1o_ref[...],
                preferred_element_type=jnp.float32)
    x += jnp.dot(nbd, wm1n_ref[...], preferred_element_type=jnp.float32)
    x = jnp.maximum(x + bm1_ref[...], 0.0)
    x = jnp.maximum(jnp.dot(x, wm2_ref[...],
                            preferred_element_type=jnp.float32)
                    + bm2_ref[...], 0.0)

    logits = jnp.dot(x, wa_ref[...],
                     preferred_element_type=jnp.float32) + ba_ref[...]
    m = jnp.max(logits, axis=-1, keepdims=True)
    lse0 = jnp.log(jnp.sum(jnp.exp(logits - m), axis=-1, keepdims=True))
    a_iota = lax.broadcasted_iota(jnp.int32, (BB, A), 1)
    cand = jnp.where(logits == m, a_iota, A)
    act_ref[...] = jnp.min(cand, axis=-1, keepdims=True)
    alp_ref[...] = 0.0 - lse0


@jax.jit
def kernel(obs, node_obs, adj, agent_id,
           W_gnn1, b_gnn1, W_gnn2, b_gnn2,
           W_mlp1, b_mlp1, W_mlp2, b_mlp2,
           W_act, b_act):
    w1x = jnp.concatenate([W_gnn1, jnp.zeros((F, H), jnp.float32)], axis=1)
    b1x = jnp.concatenate([b_gnn1, jnp.ones((H,), jnp.float32)])
    w2x = jnp.block([[W_gnn2, jnp.zeros((H, H), jnp.float32)],
                     [jnp.zeros((H, H), jnp.float32), jnp.eye(H, dtype=jnp.float32)]])
    w_mlp1_obs = W_mlp1[:OBS]
    w_mlp1_nbd = W_mlp1[OBS:]
    b1xr = b1x.reshape(1, 1, 2 * H)
    b2 = b_gnn2.reshape(1, 1, H)
    bm1 = b_mlp1.reshape(1, H)
    bm2 = b_mlp2.reshape(1, H)
    ba = b_act.reshape(1, A)

    grid = (B // BB,)
    actions, alp = pl.pallas_call(
        _fused_kernel,
        grid=grid,
        in_specs=[
            pl.BlockSpec((BB, OBS), lambda i: (i, 0)),
            pl.BlockSpec((BB, N, F), lambda i: (i, 0, 0)),
            pl.BlockSpec((BB, N, N), lambda i: (i, 0, 0)),
            pl.BlockSpec((BB, 1), lambda i: (i, 0)),
            pl.BlockSpec((F, 2 * H), lambda i: (0, 0)),
            pl.BlockSpec((1, 1, 2 * H), lambda i: (0, 0, 0)),
            pl.BlockSpec((2 * H, 2 * H), lambda i: (0, 0)),
            pl.BlockSpec((1, 1, H), lambda i: (0, 0, 0)),
            pl.BlockSpec((OBS, H), lambda i: (0, 0)),
            pl.BlockSpec((H, H), lambda i: (0, 0)),
            pl.BlockSpec((1, H), lambda i: (0, 0)),
            pl.BlockSpec((H, H), lambda i: (0, 0)),
            pl.BlockSpec((1, H), lambda i: (0, 0)),
            pl.BlockSpec((H, A), lambda i: (0, 0)),
            pl.BlockSpec((1, A), lambda i: (0, 0)),
        ],
        out_specs=[
            pl.BlockSpec((BB, 1), lambda i: (i, 0)),
            pl.BlockSpec((BB, 1), lambda i: (i, 0)),
        ],
        out_shape=[
            jax.ShapeDtypeStruct((B, 1), jnp.int32),
            jax.ShapeDtypeStruct((B, 1), jnp.float32),
        ],
        compiler_params=pltpu.CompilerParams(
            dimension_semantics=("arbitrary",),
        ),
    )(obs, node_obs, adj, agent_id,
      w1x, b1xr, w2x, b2,
      w_mlp1_obs, w_mlp1_nbd, bm1, W_mlp2, bm2,
      W_act, ba)
    return actions, alp


# speedup vs baseline: 1.5627x; 1.0792x over previous
import functools

import jax
import jax.numpy as jnp
from jax import lax
from jax.experimental import pallas as pl
from jax.experimental.pallas import tpu as pltpu

B = 16384
N = 50
F = 16
OBS = 48
H = 64
A = 5

BB = 256
NP = 56


def _fused_kernel(obs_ref, node_ref, adj_ref, aid_ref,
                  w1_ref, b1_ref, w2_ref, b2_ref,
                  wm1o_ref, wm1n_ref, bm1_ref, wm2_ref, bm2_ref,
                  wa_ref, ba_ref,
                  act_ref, alp_ref):
    row3 = lax.broadcasted_iota(jnp.int32, (BB, NP, NP), 1)
    col3 = lax.broadcasted_iota(jnp.int32, (BB, NP, NP), 2)
    adj = jnp.where((row3 < N) & (col3 < N),
                    jnp.pad(adj_ref[...], ((0, 0), (0, NP - N), (0, NP - N))),
                    0.0)
    rowf = lax.broadcasted_iota(jnp.int32, (BB, NP, F), 1)
    node = jnp.where(rowf < N,
                     jnp.pad(node_ref[...], ((0, 0), (0, NP - N), (0, 0))),
                     0.0)

    adj_n = adj / (jnp.sum(adj, axis=-1, keepdims=True) + 1e-6)

    h1 = lax.dot_general(node, w1_ref[...], (((2,), (0,)), ((), ())),
                         preferred_element_type=jnp.float32)
    h1 = jnp.maximum(h1 + b1_ref[...], 0.0)
    g1 = lax.dot_general(adj_n, h1, (((2,), (1,)), ((0,), (0,))),
                         preferred_element_type=jnp.float32)

    h2 = lax.dot_general(g1, w2_ref[...], (((2,), (0,)), ((), ())),
                         preferred_element_type=jnp.float32)
    h2 = jnp.maximum(h2 + b2_ref[...], 0.0)

    aid = aid_ref[...]
    n_iota3 = lax.broadcasted_iota(jnp.int32, (BB, NP, NP), 1)
    mask3 = n_iota3 == aid[:, :, None]
    egon = jnp.sum(jnp.where(mask3, adj_n, 0.0), axis=1)
    nbd = lax.dot_general(egon, h2, (((1,), (1,)), ((0,), (0,))),
                          preferred_element_type=jnp.float32)

    x = jnp.dot(obs_ref[...], wm1o_ref[...],
                preferred_element_type=jnp.float32)
    x += jnp.dot(nbd, wm1n_ref[...], preferred_element_type=jnp.float32)
    x = jnp.maximum(x + bm1_ref[...], 0.0)
    x = jnp.maximum(jnp.dot(x, wm2_ref[...],
                            preferred_element_type=jnp.float32)
                    + bm2_ref[...], 0.0)

    logits = jnp.dot(x, wa_ref[...],
                     preferred_element_type=jnp.float32) + ba_ref[...]
    m = jnp.max(logits, axis=-1, keepdims=True)
    lse0 = jnp.log(jnp.sum(jnp.exp(logits - m), axis=-1, keepdims=True))
    a_iota = lax.broadcasted_iota(jnp.int32, (BB, A), 1)
    cand = jnp.where(logits == m, a_iota, A)
    act_ref[...] = jnp.min(cand, axis=-1, keepdims=True)
    alp_ref[...] = 0.0 - lse0


@jax.jit
def kernel(obs, node_obs, adj, agent_id,
           W_gnn1, b_gnn1, W_gnn2, b_gnn2,
           W_mlp1, b_mlp1, W_mlp2, b_mlp2,
           W_act, b_act):
    w_mlp1_obs = W_mlp1[:OBS]
    w_mlp1_nbd = W_mlp1[OBS:]
    b1 = b_gnn1.reshape(1, 1, H)
    b2 = b_gnn2.reshape(1, 1, H)
    bm1 = b_mlp1.reshape(1, H)
    bm2 = b_mlp2.reshape(1, H)
    ba = b_act.reshape(1, A)

    grid = (B // BB,)
    actions, alp = pl.pallas_call(
        _fused_kernel,
        grid=grid,
        in_specs=[
            pl.BlockSpec((BB, OBS), lambda i: (i, 0)),
            pl.BlockSpec((BB, N, F), lambda i: (i, 0, 0)),
            pl.BlockSpec((BB, N, N), lambda i: (i, 0, 0)),
            pl.BlockSpec((BB, 1), lambda i: (i, 0)),
            pl.BlockSpec((F, H), lambda i: (0, 0)),
            pl.BlockSpec((1, 1, H), lambda i: (0, 0, 0)),
            pl.BlockSpec((H, H), lambda i: (0, 0)),
            pl.BlockSpec((1, 1, H), lambda i: (0, 0, 0)),
            pl.BlockSpec((OBS, H), lambda i: (0, 0)),
            pl.BlockSpec((H, H), lambda i: (0, 0)),
            pl.BlockSpec((1, H), lambda i: (0, 0)),
            pl.BlockSpec((H, H), lambda i: (0, 0)),
            pl.BlockSpec((1, H), lambda i: (0, 0)),
            pl.BlockSpec((H, A), lambda i: (0, 0)),
            pl.BlockSpec((1, A), lambda i: (0, 0)),
        ],
        out_specs=[
            pl.BlockSpec((BB, 1), lambda i: (i, 0)),
            pl.BlockSpec((BB, 1), lambda i: (i, 0)),
        ],
        out_shape=[
            jax.ShapeDtypeStruct((B, 1), jnp.int32),
            jax.ShapeDtypeStruct((B, 1), jnp.float32),
        ],
        compiler_params=pltpu.CompilerParams(
            dimension_semantics=("arbitrary",),
        ),
    )(obs, node_obs, adj, agent_id,
      W_gnn1, b1, W_gnn2, b2,
      w_mlp1_obs, w_mlp1_nbd, bm1, W_mlp2, bm2,
      W_act, ba)
    return actions, alp


# speedup vs baseline: 1.8071x; 1.1564x over previous
import functools

import jax
import jax.numpy as jnp
from jax import lax
from jax.experimental import pallas as pl
from jax.experimental.pallas import tpu as pltpu

B = 16384
N = 50
F = 16
OBS = 48
H = 64
A = 5

BB = 256
NP = 56


def _fused_kernel(obs_ref, node_ref, adj_ref, aid_ref,
                  w1_ref, b1_ref, w2_ref, b2_ref,
                  wm1o_ref, wm1n_ref, bm1_ref, wm2_ref, bm2_ref,
                  wa_ref, ba_ref,
                  act_ref, alp_ref):
    row3 = lax.broadcasted_iota(jnp.int32, (BB, NP, NP), 1)
    col3 = lax.broadcasted_iota(jnp.int32, (BB, NP, NP), 2)
    adj = jnp.where((row3 < N) & (col3 < N),
                    jnp.pad(adj_ref[...], ((0, 0), (0, NP - N), (0, NP - N))),
                    0.0)
    colf = lax.broadcasted_iota(jnp.int32, (BB, F, NP), 2)
    node_t = jnp.where(colf < N,
                       jnp.pad(node_ref[...], ((0, 0), (0, 0), (0, NP - N))),
                       0.0)

    adj_n = adj / (jnp.sum(adj, axis=-1, keepdims=True) + 1e-6)

    h1 = lax.dot_general(node_t, w1_ref[...], (((1,), (0,)), ((), ())),
                         preferred_element_type=jnp.float32)
    h1 = jnp.maximum(h1 + b1_ref[...], 0.0)
    g1 = lax.dot_general(adj_n, h1, (((2,), (1,)), ((0,), (0,))),
                         preferred_element_type=jnp.float32)

    h2 = lax.dot_general(g1, w2_ref[...], (((2,), (0,)), ((), ())),
                         preferred_element_type=jnp.float32)
    h2 = jnp.maximum(h2 + b2_ref[...], 0.0)

    aid = aid_ref[...]
    n_iota3 = lax.broadcasted_iota(jnp.int32, (BB, NP, NP), 1)
    mask3 = n_iota3 == aid[:, :, None]
    egon = jnp.sum(jnp.where(mask3, adj_n, 0.0), axis=1)
    nbd = lax.dot_general(egon, h2, (((1,), (1,)), ((0,), (0,))),
                          preferred_element_type=jnp.float32)

    x = jnp.dot(obs_ref[...], wm1o_ref[...],
                preferred_element_type=jnp.float32)
    x += jnp.dot(nbd, wm1n_ref[...], preferred_element_type=jnp.float32)
    x = jnp.maximum(x + bm1_ref[...], 0.0)
    x = jnp.maximum(jnp.dot(x, wm2_ref[...],
                            preferred_element_type=jnp.float32)
                    + bm2_ref[...], 0.0)

    logits = jnp.dot(x, wa_ref[...],
                     preferred_element_type=jnp.float32) + ba_ref[...]
    m = jnp.max(logits, axis=-1, keepdims=True)
    lse0 = jnp.log(jnp.sum(jnp.exp(logits - m), axis=-1, keepdims=True))
    a_iota = lax.broadcasted_iota(jnp.int32, (BB, A), 1)
    cand = jnp.where(logits == m, a_iota, A)
    act_ref[...] = jnp.min(cand, axis=-1, keepdims=True)
    alp_ref[...] = 0.0 - lse0


@jax.jit
def kernel(obs, node_obs, adj, agent_id,
           W_gnn1, b_gnn1, W_gnn2, b_gnn2,
           W_mlp1, b_mlp1, W_mlp2, b_mlp2,
           W_act, b_act):
    w_mlp1_obs = W_mlp1[:OBS]
    w_mlp1_nbd = W_mlp1[OBS:]
    b1 = b_gnn1.reshape(1, 1, H)
    b2 = b_gnn2.reshape(1, 1, H)
    bm1 = b_mlp1.reshape(1, H)
    bm2 = b_mlp2.reshape(1, H)
    ba = b_act.reshape(1, A)

    grid = (B // BB,)
    actions, alp = pl.pallas_call(
        _fused_kernel,
        grid=grid,
        in_specs=[
            pl.BlockSpec((BB, OBS), lambda i: (i, 0)),
            pl.BlockSpec((BB, F, N), lambda i: (i, 0, 0)),
            pl.BlockSpec((BB, N, N), lambda i: (i, 0, 0)),
            pl.BlockSpec((BB, 1), lambda i: (i, 0)),
            pl.BlockSpec((F, H), lambda i: (0, 0)),
            pl.BlockSpec((1, 1, H), lambda i: (0, 0, 0)),
            pl.BlockSpec((H, H), lambda i: (0, 0)),
            pl.BlockSpec((1, 1, H), lambda i: (0, 0, 0)),
            pl.BlockSpec((OBS, H), lambda i: (0, 0)),
            pl.BlockSpec((H, H), lambda i: (0, 0)),
            pl.BlockSpec((1, H), lambda i: (0, 0)),
            pl.BlockSpec((H, H), lambda i: (0, 0)),
            pl.BlockSpec((1, H), lambda i: (0, 0)),
            pl.BlockSpec((H, A), lambda i: (0, 0)),
            pl.BlockSpec((1, A), lambda i: (0, 0)),
        ],
        out_specs=[
            pl.BlockSpec((BB, 1), lambda i: (i, 0)),
            pl.BlockSpec((BB, 1), lambda i: (i, 0)),
        ],
        out_shape=[
            jax.ShapeDtypeStruct((B, 1), jnp.int32),
            jax.ShapeDtypeStruct((B, 1), jnp.float32),
        ],
        compiler_params=pltpu.CompilerParams(
            dimension_semantics=("arbitrary",),
        ),
    )(obs, jnp.swapaxes(node_obs, 1, 2), adj, agent_id,
      W_gnn1, b1, W_gnn2, b2,
      w_mlp1_obs, w_mlp1_nbd, bm1, W_mlp2, bm2,
      W_act, ba)
    return actions, alp
